# Initial kernel scaffold; baseline (speedup 1.0000x reference)
import jax
import jax.numpy as jnp
from jax.experimental import pallas as pl


def kernel(pc, W_in, b_in, Wd0_0, bd0_0, Wd0_1, bd0_1, Wd0_2, bd0_2, Wc1, bc1, Wd1_0, bd1_0, Wd1_1, bd1_1, Wd1_2, bd1_2, Wc2, bc2, Wd2_0, bd2_0, Wd2_1, bd2_1, Wd2_2, bd2_2, Wc3, bc3, Wd3_0, bd3_0, Wd3_1, bd3_1, Wd3_2, bd3_2):
    raise NotImplementedError("write your pallas kernel here")


# speedup vs baseline: 9.0441x; 9.0441x over previous
import functools

import jax
import jax.numpy as jnp
import numpy as np
from jax.experimental import pallas as pl
from jax.experimental.pallas import tpu as pltpu
from jax.experimental.pallas import tpu_sc as plsc

N = 2048
B = 4
K = 16
M = 256
CP = 32
TW = 128
LANES = K * CP


def _dot(a, b):
    return jax.lax.dot_general(a, b, (((1,), (0,)), ((), ())),
                               preferred_element_type=jnp.float32)


def _pad_rc(w, rows, cols):
    return jnp.pad(w, ((0, rows - w.shape[0]), (0, cols - w.shape[1])))


def _l0_body(pc_ref, w_ref, b_ref, o_ref):
    o_ref[0] = jax.nn.relu(_dot(pc_ref[0], w_ref[...]) + b_ref[...])


def _l0_call(pc, W_in, b_in):
    return pl.pallas_call(
        _l0_body,
        grid=(B,),
        in_specs=[
            pl.BlockSpec((1, N, 3), lambda b: (b, 0, 0)),
            pl.BlockSpec((3, 24), lambda b: (0, 0)),
            pl.BlockSpec((1, 24), lambda b: (0, 0)),
        ],
        out_specs=pl.BlockSpec((1, N, 24), lambda b: (b, 0, 0)),
        out_shape=jax.ShapeDtypeStruct((B, N, 24), jnp.float32),
    )(pc, W_in, b_in.reshape(1, 24))


def _knn_body(f_ref, ft_ref, w0a_ref, cw0_ref, cw1_ref, cw2_ref,
              b0_ref, b1_ref, b2_ref,
              gidx_ref, a_ref, c0_ref, c1_ref, c2_ref):
    b = pl.program_id(0)
    f = f_ref[0]
    ft = ft_ref[0]
    dot = _dot(f, ft)
    sq_all = jnp.sum(ft * ft, axis=0, keepdims=True)
    sq_t = jnp.sum(f * f, axis=1, keepdims=True)
    D = sq_t - 2.0 * dot + sq_all

    col = jax.lax.broadcasted_iota(jnp.int32, (M, N), 1)
    big = jnp.int32(2 ** 30)
    inf = jnp.float32(3e38)
    cols = []
    for _ in range(K):
        mn = jnp.min(D, axis=1, keepdims=True)
        amn = jnp.min(jnp.where(D == mn, col, big), axis=1)
        cols.append(amn[:, None])
        D = jnp.where(col == amn[:, None], inf, D)
    idx = jnp.concatenate(cols, axis=1)
    gidx_ref[0] = idx + b * N

    a_ref[0] = _dot(f, w0a_ref[...])
    c0_ref[0] = _dot(f, cw0_ref[...]) + b0_ref[...]
    c1_ref[0] = _dot(f, cw1_ref[...]) + b1_ref[...]
    c2_ref[0] = _dot(f, cw2_ref[...]) + b2_ref[...]


def _knn_call(f, ft, w0a, cw0, cw1, cw2, b0, b1, b2):
    C = f.shape[-1]
    wspec = pl.BlockSpec((C, CP), lambda b, i: (0, 0))
    bspec = pl.BlockSpec((1, CP), lambda b, i: (0, 0))
    ospec = pl.BlockSpec((1, M, CP), lambda b, i: (b, i, 0))
    return pl.pallas_call(
        _knn_body,
        grid=(B, N // M),
        in_specs=[
            pl.BlockSpec((1, M, C), lambda b, i: (b, i, 0)),
            pl.BlockSpec((1, C, N), lambda b, i: (b, 0, 0)),
            pl.BlockSpec((C, TW), lambda b, i: (0, 0)),
            wspec, wspec, wspec, bspec, bspec, bspec,
        ],
        out_specs=[
            pl.BlockSpec((1, M, K), lambda b, i: (b, i, 0)),
            pl.BlockSpec((1, M, TW), lambda b, i: (b, i, 0)),
            ospec, ospec, ospec,
        ],
        out_shape=[
            jax.ShapeDtypeStruct((B, N, K), jnp.int32),
            jax.ShapeDtypeStruct((B, N, TW), jnp.float32),
            jax.ShapeDtypeStruct((B, N, CP), jnp.float32),
            jax.ShapeDtypeStruct((B, N, CP), jnp.float32),
            jax.ShapeDtypeStruct((B, N, CP), jnp.float32),
        ],
    )(f, ft, w0a, cw0, cw1, cw2, b0, b1, b2)


_GW = 128
_NIDX = B * N * K


def _sc_gather(table, indices):
    mesh = plsc.VectorSubcoreMesh(core_axis_name="core",
                                  subcore_axis_name="subcore")

    @functools.partial(
        pl.kernel,
        out_type=jax.ShapeDtypeStruct((_NIDX, TW), jnp.float32),
        mesh=mesh,
    )
    def gather_kernel(x_hbm, i_hbm, o_hbm):
        def body(i_vmem, o_vmem):
            pltpu.sync_copy(x_hbm.at[i_vmem.at[0]], o_vmem)

        pltpu.emit_pipeline(
            body,
            grid=(_NIDX // _GW,),
            in_specs=[pl.BlockSpec((1, _GW), index_map=lambda i: (0, i))],
            out_specs=[pl.BlockSpec((_GW, TW), index_map=lambda i: (i, 0))],
            core_axis_name=("core", "subcore"),
            dimension_semantics=(pltpu.PARALLEL,),
        )(i_hbm, o_hbm)

    return gather_kernel(table, indices)


def _rep16(c):
    return jnp.concatenate([c] * K, axis=1)


def _lane_max16(y):
    a = jnp.maximum(y[:, :256], y[:, 256:])
    a = jnp.maximum(a[:, :128], a[:, 128:])
    a = jnp.maximum(a[:, :64], a[:, 64:])
    return jnp.maximum(a[:, :32], a[:, 32:])


def _edge_stages(ag_ref, c0_ref, c1_ref, c2_ref, w1_ref, w2a_ref, w2b_ref):
    agw = ag_ref[0]
    ag = jnp.concatenate([agw[:, j * TW:j * TW + CP] for j in range(K)],
                         axis=1)
    y0 = jax.nn.relu(ag + _rep16(c0_ref[0]))
    y1 = jax.nn.relu(_dot(y0, w1_ref[...]) + _rep16(c1_ref[0]))
    y2 = jax.nn.relu(_dot(y1, w2a_ref[...]) + _dot(y0, w2b_ref[...])
                     + _rep16(c2_ref[0]))
    return _lane_max16(y0), _lane_max16(y1), _lane_max16(y2)


def _edge_body_t(ag_ref, c0_ref, c1_ref, c2_ref, rest_ref,
                 w1_ref, w2a_ref, w2b_ref,
                 wca2_ref, wca1_ref, wca0_ref, wcb_ref, bc_ref,
                 m2_ref, m1_ref, m0_ref, t_ref):
    m0, m1, m2 = _edge_stages(ag_ref, c0_ref, c1_ref, c2_ref,
                              w1_ref, w2a_ref, w2b_ref)
    m0_ref[0], m1_ref[0], m2_ref[0] = m0, m1, m2
    t_ref[0] = jax.nn.relu(
        _dot(m2, wca2_ref[...]) + _dot(m1, wca1_ref[...])
        + _dot(m0, wca0_ref[...]) + _dot(rest_ref[0], wcb_ref[...])
        + bc_ref[...])


def _edge_body(ag_ref, c0_ref, c1_ref, c2_ref,
               w1_ref, w2a_ref, w2b_ref,
               m2_ref, m1_ref, m0_ref):
    m0, m1, m2 = _edge_stages(ag_ref, c0_ref, c1_ref, c2_ref,
                              w1_ref, w2a_ref, w2b_ref)
    m0_ref[0], m1_ref[0], m2_ref[0] = m0, m1, m2


def _edge_call(ag, c0, c1, c2, w1, w2a, w2b, rest=None, wca=None, wcb=None,
               bc=None):
    cspec = pl.BlockSpec((1, M, CP), lambda b, i: (b, i, 0))
    wspec = pl.BlockSpec((LANES, LANES), lambda b, i: (0, 0))
    in_specs = [
        pl.BlockSpec((1, M, K * TW), lambda b, i: (b, i, 0)),
        cspec, cspec, cspec,
    ]
    mspec = pl.BlockSpec((1, M, CP), lambda b, i: (b, i, 0))
    mshape = jax.ShapeDtypeStruct((B, N, CP), jnp.float32)
    if rest is None:
        return pl.pallas_call(
            _edge_body,
            grid=(B, N // M),
            in_specs=in_specs + [wspec, wspec, wspec],
            out_specs=[mspec, mspec, mspec],
            out_shape=[mshape, mshape, mshape],
        )(ag, c0, c1, c2, w1, w2a, w2b)
    R = rest.shape[-1]
    wcaspec = pl.BlockSpec((CP, 48), lambda b, i: (0, 0))
    return pl.pallas_call(
        _edge_body_t,
        grid=(B, N // M),
        in_specs=in_specs + [
            pl.BlockSpec((1, M, R), lambda b, i: (b, i, 0)),
            wspec, wspec, wspec,
            wcaspec, wcaspec, wcaspec,
            pl.BlockSpec((R, 48), lambda b, i: (0, 0)),
            pl.BlockSpec((1, 48), lambda b, i: (0, 0)),
        ],
        out_specs=[mspec, mspec, mspec,
                   pl.BlockSpec((1, M, 48), lambda b, i: (b, i, 0))],
        out_shape=[mshape, mshape, mshape,
                   jax.ShapeDtypeStruct((B, N, 48), jnp.float32)],
    )(ag, c0, c1, c2, rest, w1, w2a, w2b, wca[0], wca[1], wca[2],
      wcb, bc.reshape(1, 48))


def _block(f, l_prev, W0, b0, W1, b1, W2, b2, Wc=None, bc=None):
    C = f.shape[-1]
    w0a = _pad_rc(W0[:C], C, TW)
    cw0 = _pad_rc(W0[C:] - W0[:C], C, CP)
    cw1 = _pad_rc(W1[24:], C, CP)
    cw2 = _pad_rc(W2[48:], C, CP)
    b0p = _pad_rc(b0.reshape(1, 24), 1, CP)
    b1p = _pad_rc(b1.reshape(1, 24), 1, CP)
    b2p = _pad_rc(b2.reshape(1, 24), 1, CP)

    ft = jnp.swapaxes(f, 1, 2)
    gidx, A, c0, c1, c2 = _knn_call(f, ft, w0a, cw0, cw1, cw2, b0p, b1p, b2p)

    ag = _sc_gather(A.reshape(B * N, TW), gidx.reshape(1, _NIDX))
    ag = ag.reshape(B, N, K * TW)

    eye = jnp.eye(K, dtype=jnp.float32)
    w1bd = jnp.kron(eye, _pad_rc(W1[:24], CP, CP))
    w2abd = jnp.kron(eye, _pad_rc(W2[:24], CP, CP))
    w2bbd = jnp.kron(eye, _pad_rc(W2[24:48], CP, CP))

    if Wc is None:
        m2, m1, m0 = _edge_call(ag, c0, c1, c2, w1bd, w2abd, w2bbd)
        t = None
    else:
        rest = jnp.concatenate([f, l_prev], axis=-1)
        wca = [_pad_rc(Wc[0:24], CP, 48), _pad_rc(Wc[24:48], CP, 48),
               _pad_rc(Wc[48:72], CP, 48)]
        wcb = Wc[72:]
        m2, m1, m0, t = _edge_call(ag, c0, c1, c2, w1bd, w2abd, w2bbd,
                                   rest=rest, wca=wca, wcb=wcb, bc=bc)
    maxes = jnp.concatenate([m2[..., :24], m1[..., :24], m0[..., :24]],
                            axis=-1)
    return maxes, t


def kernel(pc, W_in, b_in, Wd0_0, bd0_0, Wd0_1, bd0_1, Wd0_2, bd0_2,
           Wc1, bc1, Wd1_0, bd1_0, Wd1_1, bd1_1, Wd1_2, bd1_2,
           Wc2, bc2, Wd2_0, bd2_0, Wd2_1, bd2_1, Wd2_2, bd2_2,
           Wc3, bc3, Wd3_0, bd3_0, Wd3_1, bd3_1, Wd3_2, bd3_2):
    l0 = _l0_call(pc, W_in, b_in)

    mx1, t1 = _block(l0, l0, Wd0_0, bd0_0, Wd0_1, bd0_1, Wd0_2, bd0_2,
                     Wc=Wc1, bc=bc1)
    l1 = jnp.concatenate([mx1, l0, l0], axis=-1)

    mx2, t2 = _block(t1, l1, Wd1_0, bd1_0, Wd1_1, bd1_1, Wd1_2, bd1_2,
                     Wc=Wc2, bc=bc2)
    l2 = jnp.concatenate([mx2, t1, l1], axis=-1)

    mx3, t3 = _block(t2, l2, Wd2_0, bd2_0, Wd2_1, bd2_1, Wd2_2, bd2_2,
                     Wc=Wc3, bc=bc3)
    l3 = jnp.concatenate([mx3, t2, l2], axis=-1)

    mx4, _ = _block(t3, l3, Wd3_0, bd3_0, Wd3_1, bd3_1, Wd3_2, bd3_2)
    return jnp.concatenate([mx4, t3, l3], axis=-1)


# speedup vs baseline: 9.0511x; 1.0008x over previous
import functools

import jax
import jax.numpy as jnp
import numpy as np
from jax.experimental import pallas as pl
from jax.experimental.pallas import tpu as pltpu
from jax.experimental.pallas import tpu_sc as plsc

N = 2048
B = 4
K = 16
M = 256
CP = 32
TW = 128
LANES = K * CP


_TC_PARAMS = pltpu.CompilerParams(
    dimension_semantics=("parallel", "arbitrary"))


def _dot(a, b):
    return jax.lax.dot_general(a, b, (((1,), (0,)), ((), ())),
                               preferred_element_type=jnp.float32)


def _pad_rc(w, rows, cols):
    return jnp.pad(w, ((0, rows - w.shape[0]), (0, cols - w.shape[1])))


def _l0_body(pc_ref, w_ref, b_ref, o_ref):
    o_ref[0] = jax.nn.relu(_dot(pc_ref[0], w_ref[...]) + b_ref[...])


def _l0_call(pc, W_in, b_in):
    return pl.pallas_call(
        _l0_body,
        grid=(B,),
        in_specs=[
            pl.BlockSpec((1, N, 3), lambda b: (b, 0, 0)),
            pl.BlockSpec((3, 24), lambda b: (0, 0)),
            pl.BlockSpec((1, 24), lambda b: (0, 0)),
        ],
        out_specs=pl.BlockSpec((1, N, 24), lambda b: (b, 0, 0)),
        out_shape=jax.ShapeDtypeStruct((B, N, 24), jnp.float32),
        compiler_params=pltpu.CompilerParams(
            dimension_semantics=("parallel",)),
    )(pc, W_in, b_in.reshape(1, 24))


def _knn_body(f_ref, ft_ref, w0a_ref, cw0_ref, cw1_ref, cw2_ref,
              b0_ref, b1_ref, b2_ref,
              gidx_ref, a_ref, c0_ref, c1_ref, c2_ref):
    b = pl.program_id(0)
    f = f_ref[0]
    ft = ft_ref[0]
    dot = _dot(f, ft)
    sq_all = jnp.sum(ft * ft, axis=0, keepdims=True)
    sq_t = jnp.sum(f * f, axis=1, keepdims=True)
    D = sq_t - 2.0 * dot + sq_all

    col = jax.lax.broadcasted_iota(jnp.int32, (M, N), 1)
    big = jnp.int32(2 ** 30)
    inf = jnp.float32(3e38)
    cols = []
    for _ in range(K):
        mn = jnp.min(D, axis=1, keepdims=True)
        amn = jnp.min(jnp.where(D == mn, col, big), axis=1)
        cols.append(amn[:, None])
        D = jnp.where(col == amn[:, None], inf, D)
    idx = jnp.concatenate(cols, axis=1)
    gidx_ref[0] = idx + b * N

    a_ref[0] = _dot(f, w0a_ref[...])
    c0_ref[0] = _dot(f, cw0_ref[...]) + b0_ref[...]
    c1_ref[0] = _dot(f, cw1_ref[...]) + b1_ref[...]
    c2_ref[0] = _dot(f, cw2_ref[...]) + b2_ref[...]


def _knn_call(f, ft, w0a, cw0, cw1, cw2, b0, b1, b2):
    C = f.shape[-1]
    wspec = pl.BlockSpec((C, CP), lambda b, i: (0, 0))
    bspec = pl.BlockSpec((1, CP), lambda b, i: (0, 0))
    ospec = pl.BlockSpec((1, M, CP), lambda b, i: (b, i, 0))
    return pl.pallas_call(
        _knn_body,
        grid=(B, N // M),
        in_specs=[
            pl.BlockSpec((1, M, C), lambda b, i: (b, i, 0)),
            pl.BlockSpec((1, C, N), lambda b, i: (b, 0, 0)),
            pl.BlockSpec((C, TW), lambda b, i: (0, 0)),
            wspec, wspec, wspec, bspec, bspec, bspec,
        ],
        out_specs=[
            pl.BlockSpec((1, M, K), lambda b, i: (b, i, 0)),
            pl.BlockSpec((1, M, TW), lambda b, i: (b, i, 0)),
            ospec, ospec, ospec,
        ],
        out_shape=[
            jax.ShapeDtypeStruct((B, N, K), jnp.int32),
            jax.ShapeDtypeStruct((B, N, TW), jnp.float32),
            jax.ShapeDtypeStruct((B, N, CP), jnp.float32),
            jax.ShapeDtypeStruct((B, N, CP), jnp.float32),
            jax.ShapeDtypeStruct((B, N, CP), jnp.float32),
        ],
        compiler_params=_TC_PARAMS,
    )(f, ft, w0a, cw0, cw1, cw2, b0, b1, b2)


_GW = 128
_NIDX = B * N * K


def _sc_gather(table, indices):
    mesh = plsc.VectorSubcoreMesh(core_axis_name="core",
                                  subcore_axis_name="subcore")

    @functools.partial(
        pl.kernel,
        out_type=jax.ShapeDtypeStruct((_NIDX, TW), jnp.float32),
        mesh=mesh,
    )
    def gather_kernel(x_hbm, i_hbm, o_hbm):
        def body(i_vmem, o_vmem):
            pltpu.sync_copy(x_hbm.at[i_vmem.at[0]], o_vmem)

        pltpu.emit_pipeline(
            body,
            grid=(_NIDX // _GW,),
            in_specs=[pl.BlockSpec((1, _GW), index_map=lambda i: (0, i))],
            out_specs=[pl.BlockSpec((_GW, TW), index_map=lambda i: (i, 0))],
            core_axis_name=("core", "subcore"),
            dimension_semantics=(pltpu.PARALLEL,),
        )(i_hbm, o_hbm)

    return gather_kernel(table, indices)


def _rep16(c):
    return jnp.concatenate([c] * K, axis=1)


def _lane_max16(y):
    a = jnp.maximum(y[:, :256], y[:, 256:])
    a = jnp.maximum(a[:, :128], a[:, 128:])
    a = jnp.maximum(a[:, :64], a[:, 64:])
    return jnp.maximum(a[:, :32], a[:, 32:])


def _edge_stages(ag_ref, c0_ref, c1_ref, c2_ref, w1_ref, w2a_ref, w2b_ref):
    agw = ag_ref[0]
    ag = jnp.concatenate([agw[:, j * TW:j * TW + CP] for j in range(K)],
                         axis=1)
    y0 = jax.nn.relu(ag + _rep16(c0_ref[0]))
    y1 = jax.nn.relu(_dot(y0, w1_ref[...]) + _rep16(c1_ref[0]))
    y2 = jax.nn.relu(_dot(y1, w2a_ref[...]) + _dot(y0, w2b_ref[...])
                     + _rep16(c2_ref[0]))
    return _lane_max16(y0), _lane_max16(y1), _lane_max16(y2)


def _edge_body_t(ag_ref, c0_ref, c1_ref, c2_ref, rest_ref,
                 w1_ref, w2a_ref, w2b_ref,
                 wca2_ref, wca1_ref, wca0_ref, wcb_ref, bc_ref,
                 m2_ref, m1_ref, m0_ref, t_ref):
    m0, m1, m2 = _edge_stages(ag_ref, c0_ref, c1_ref, c2_ref,
                              w1_ref, w2a_ref, w2b_ref)
    m0_ref[0], m1_ref[0], m2_ref[0] = m0, m1, m2
    t_ref[0] = jax.nn.relu(
        _dot(m2, wca2_ref[...]) + _dot(m1, wca1_ref[...])
        + _dot(m0, wca0_ref[...]) + _dot(rest_ref[0], wcb_ref[...])
        + bc_ref[...])


def _edge_body(ag_ref, c0_ref, c1_ref, c2_ref,
               w1_ref, w2a_ref, w2b_ref,
               m2_ref, m1_ref, m0_ref):
    m0, m1, m2 = _edge_stages(ag_ref, c0_ref, c1_ref, c2_ref,
                              w1_ref, w2a_ref, w2b_ref)
    m0_ref[0], m1_ref[0], m2_ref[0] = m0, m1, m2


def _edge_call(ag, c0, c1, c2, w1, w2a, w2b, rest=None, wca=None, wcb=None,
               bc=None):
    cspec = pl.BlockSpec((1, M, CP), lambda b, i: (b, i, 0))
    wspec = pl.BlockSpec((LANES, LANES), lambda b, i: (0, 0))
    in_specs = [
        pl.BlockSpec((1, M, K * TW), lambda b, i: (b, i, 0)),
        cspec, cspec, cspec,
    ]
    mspec = pl.BlockSpec((1, M, CP), lambda b, i: (b, i, 0))
    mshape = jax.ShapeDtypeStruct((B, N, CP), jnp.float32)
    if rest is None:
        return pl.pallas_call(
            _edge_body,
            grid=(B, N // M),
            in_specs=in_specs + [wspec, wspec, wspec],
            out_specs=[mspec, mspec, mspec],
            out_shape=[mshape, mshape, mshape],
            compiler_params=_TC_PARAMS,
        )(ag, c0, c1, c2, w1, w2a, w2b)
    R = rest.shape[-1]
    wcaspec = pl.BlockSpec((CP, 48), lambda b, i: (0, 0))
    return pl.pallas_call(
        _edge_body_t,
        grid=(B, N // M),
        in_specs=in_specs + [
            pl.BlockSpec((1, M, R), lambda b, i: (b, i, 0)),
            wspec, wspec, wspec,
            wcaspec, wcaspec, wcaspec,
            pl.BlockSpec((R, 48), lambda b, i: (0, 0)),
            pl.BlockSpec((1, 48), lambda b, i: (0, 0)),
        ],
        out_specs=[mspec, mspec, mspec,
                   pl.BlockSpec((1, M, 48), lambda b, i: (b, i, 0))],
        out_shape=[mshape, mshape, mshape,
                   jax.ShapeDtypeStruct((B, N, 48), jnp.float32)],
        compiler_params=_TC_PARAMS,
    )(ag, c0, c1, c2, rest, w1, w2a, w2b, wca[0], wca[1], wca[2],
      wcb, bc.reshape(1, 48))


def _block(f, l_prev, W0, b0, W1, b1, W2, b2, Wc=None, bc=None):
    C = f.shape[-1]
    w0a = _pad_rc(W0[:C], C, TW)
    cw0 = _pad_rc(W0[C:] - W0[:C], C, CP)
    cw1 = _pad_rc(W1[24:], C, CP)
    cw2 = _pad_rc(W2[48:], C, CP)
    b0p = _pad_rc(b0.reshape(1, 24), 1, CP)
    b1p = _pad_rc(b1.reshape(1, 24), 1, CP)
    b2p = _pad_rc(b2.reshape(1, 24), 1, CP)

    ft = jnp.swapaxes(f, 1, 2)
    gidx, A, c0, c1, c2 = _knn_call(f, ft, w0a, cw0, cw1, cw2, b0p, b1p, b2p)

    ag = _sc_gather(A.reshape(B * N, TW), gidx.reshape(1, _NIDX))
    ag = ag.reshape(B, N, K * TW)

    eye = jnp.eye(K, dtype=jnp.float32)
    w1bd = jnp.kron(eye, _pad_rc(W1[:24], CP, CP))
    w2abd = jnp.kron(eye, _pad_rc(W2[:24], CP, CP))
    w2bbd = jnp.kron(eye, _pad_rc(W2[24:48], CP, CP))

    if Wc is None:
        m2, m1, m0 = _edge_call(ag, c0, c1, c2, w1bd, w2abd, w2bbd)
        t = None
    else:
        rest = jnp.concatenate([f, l_prev], axis=-1)
        wca = [_pad_rc(Wc[0:24], CP, 48), _pad_rc(Wc[24:48], CP, 48),
               _pad_rc(Wc[48:72], CP, 48)]
        wcb = Wc[72:]
        m2, m1, m0, t = _edge_call(ag, c0, c1, c2, w1bd, w2abd, w2bbd,
                                   rest=rest, wca=wca, wcb=wcb, bc=bc)
    maxes = jnp.concatenate([m2[..., :24], m1[..., :24], m0[..., :24]],
                            axis=-1)
    return maxes, t


def kernel(pc, W_in, b_in, Wd0_0, bd0_0, Wd0_1, bd0_1, Wd0_2, bd0_2,
           Wc1, bc1, Wd1_0, bd1_0, Wd1_1, bd1_1, Wd1_2, bd1_2,
           Wc2, bc2, Wd2_0, bd2_0, Wd2_1, bd2_1, Wd2_2, bd2_2,
           Wc3, bc3, Wd3_0, bd3_0, Wd3_1, bd3_1, Wd3_2, bd3_2):
    l0 = _l0_call(pc, W_in, b_in)

    mx1, t1 = _block(l0, l0, Wd0_0, bd0_0, Wd0_1, bd0_1, Wd0_2, bd0_2,
                     Wc=Wc1, bc=bc1)
    l1 = jnp.concatenate([mx1, l0, l0], axis=-1)

    mx2, t2 = _block(t1, l1, Wd1_0, bd1_0, Wd1_1, bd1_1, Wd1_2, bd1_2,
                     Wc=Wc2, bc=bc2)
    l2 = jnp.concatenate([mx2, t1, l1], axis=-1)

    mx3, t3 = _block(t2, l2, Wd2_0, bd2_0, Wd2_1, bd2_1, Wd2_2, bd2_2,
                     Wc=Wc3, bc=bc3)
    l3 = jnp.concatenate([mx3, t2, l2], axis=-1)

    mx4, _ = _block(t3, l3, Wd3_0, bd3_0, Wd3_1, bd3_1, Wd3_2, bd3_2)
    return jnp.concatenate([mx4, t3, l3], axis=-1)


# speedup vs baseline: 10.8426x; 1.1979x over previous
import functools

import jax
import jax.numpy as jnp
import numpy as np
from jax.experimental import pallas as pl
from jax.experimental.pallas import tpu as pltpu
from jax.experimental.pallas import tpu_sc as plsc

N = 2048
B = 4
K = 16
M = 256
CP = 32
TW = 128
LANES = K * CP


_TC_PARAMS = pltpu.CompilerParams(
    dimension_semantics=("parallel", "arbitrary"))


def _dot(a, b):
    return jax.lax.dot_general(a, b, (((1,), (0,)), ((), ())),
                               preferred_element_type=jnp.float32)


def _pad_rc(w, rows, cols):
    return jnp.pad(w, ((0, rows - w.shape[0]), (0, cols - w.shape[1])))


def _l0_body(pc_ref, w_ref, b_ref, o_ref):
    o_ref[0] = jax.nn.relu(_dot(pc_ref[0], w_ref[...]) + b_ref[...])


def _l0_call(pc, W_in, b_in):
    return pl.pallas_call(
        _l0_body,
        grid=(B,),
        in_specs=[
            pl.BlockSpec((1, N, 3), lambda b: (b, 0, 0)),
            pl.BlockSpec((3, 24), lambda b: (0, 0)),
            pl.BlockSpec((1, 24), lambda b: (0, 0)),
        ],
        out_specs=pl.BlockSpec((1, N, 24), lambda b: (b, 0, 0)),
        out_shape=jax.ShapeDtypeStruct((B, N, 24), jnp.float32),
        compiler_params=pltpu.CompilerParams(
            dimension_semantics=("parallel",)),
    )(pc, W_in, b_in.reshape(1, 24))


def _knn_body(f_ref, ft_ref, w0a_ref, cw0_ref, cw1_ref, cw2_ref,
              b0_ref, b1_ref, b2_ref,
              gidx_ref, a_ref, c0_ref, c1_ref, c2_ref):
    b = pl.program_id(0)
    f = f_ref[0]
    ft = ft_ref[0]
    dot = _dot(f, ft)
    sq_all = jnp.sum(ft * ft, axis=0, keepdims=True)
    sq_t = jnp.sum(f * f, axis=1, keepdims=True)
    D = sq_t - 2.0 * dot + sq_all

    ng = N // 128
    dbits = jax.lax.bitcast_convert_type(jnp.maximum(D, 0.0), jnp.int32)
    dp = [(dbits[:, g * 128:(g + 1) * 128] & jnp.int32(-16)) | g
          for g in range(ng)]
    val = dp[0]
    for g in range(1, ng):
        val = jnp.minimum(val, dp[g])
    lane_iota = jax.lax.broadcasted_iota(jnp.int32, (M, 128), 1)
    big = jnp.int32(2 ** 30)
    mx = jnp.int32(2 ** 31 - 1)
    cols = []
    for _ in range(K):
        mnp = jnp.min(val, axis=1, keepdims=True)
        lane = jnp.min(jnp.where(val == mnp, lane_iota, big),
                       axis=1, keepdims=True)
        cols.append((mnp & 15) * 128 + lane)
        new_dp = []
        val = None
        for g in range(ng):
            dg = jnp.where(dp[g] == mnp, mx, dp[g])
            new_dp.append(dg)
            val = dg if val is None else jnp.minimum(val, dg)
        dp = new_dp
    idx = jnp.concatenate(cols, axis=1)
    gidx_ref[0] = idx + b * N

    a_ref[0] = _dot(f, w0a_ref[...])
    c0_ref[0] = _dot(f, cw0_ref[...]) + b0_ref[...]
    c1_ref[0] = _dot(f, cw1_ref[...]) + b1_ref[...]
    c2_ref[0] = _dot(f, cw2_ref[...]) + b2_ref[...]


def _knn_call(f, ft, w0a, cw0, cw1, cw2, b0, b1, b2):
    C = f.shape[-1]
    wspec = pl.BlockSpec((C, CP), lambda b, i: (0, 0))
    bspec = pl.BlockSpec((1, CP), lambda b, i: (0, 0))
    ospec = pl.BlockSpec((1, M, CP), lambda b, i: (b, i, 0))
    return pl.pallas_call(
        _knn_body,
        grid=(B, N // M),
        in_specs=[
            pl.BlockSpec((1, M, C), lambda b, i: (b, i, 0)),
            pl.BlockSpec((1, C, N), lambda b, i: (b, 0, 0)),
            pl.BlockSpec((C, TW), lambda b, i: (0, 0)),
            wspec, wspec, wspec, bspec, bspec, bspec,
        ],
        out_specs=[
            pl.BlockSpec((1, M, K), lambda b, i: (b, i, 0)),
            pl.BlockSpec((1, M, TW), lambda b, i: (b, i, 0)),
            ospec, ospec, ospec,
        ],
        out_shape=[
            jax.ShapeDtypeStruct((B, N, K), jnp.int32),
            jax.ShapeDtypeStruct((B, N, TW), jnp.float32),
            jax.ShapeDtypeStruct((B, N, CP), jnp.float32),
            jax.ShapeDtypeStruct((B, N, CP), jnp.float32),
            jax.ShapeDtypeStruct((B, N, CP), jnp.float32),
        ],
        compiler_params=_TC_PARAMS,
    )(f, ft, w0a, cw0, cw1, cw2, b0, b1, b2)


_GW = 128
_NIDX = B * N * K


def _sc_gather(table, indices):
    mesh = plsc.VectorSubcoreMesh(core_axis_name="core",
                                  subcore_axis_name="subcore")

    @functools.partial(
        pl.kernel,
        out_type=jax.ShapeDtypeStruct((_NIDX, TW), jnp.float32),
        mesh=mesh,
    )
    def gather_kernel(x_hbm, i_hbm, o_hbm):
        def body(i_vmem, o_vmem):
            pltpu.sync_copy(x_hbm.at[i_vmem.at[0]], o_vmem)

        pltpu.emit_pipeline(
            body,
            grid=(_NIDX // _GW,),
            in_specs=[pl.BlockSpec((1, _GW), index_map=lambda i: (0, i))],
            out_specs=[pl.BlockSpec((_GW, TW), index_map=lambda i: (i, 0))],
            core_axis_name=("core", "subcore"),
            dimension_semantics=(pltpu.PARALLEL,),
        )(i_hbm, o_hbm)

    return gather_kernel(table, indices)


def _rep16(c):
    return jnp.concatenate([c] * K, axis=1)


def _lane_max16(y):
    a = jnp.maximum(y[:, :256], y[:, 256:])
    a = jnp.maximum(a[:, :128], a[:, 128:])
    a = jnp.maximum(a[:, :64], a[:, 64:])
    return jnp.maximum(a[:, :32], a[:, 32:])


def _edge_stages(ag_ref, c0_ref, c1_ref, c2_ref, w1_ref, w2a_ref, w2b_ref):
    agw = ag_ref[0]
    ag = jnp.concatenate([agw[:, j * TW:j * TW + CP] for j in range(K)],
                         axis=1)
    y0 = jax.nn.relu(ag + _rep16(c0_ref[0]))
    y1 = jax.nn.relu(_dot(y0, w1_ref[...]) + _rep16(c1_ref[0]))
    y2 = jax.nn.relu(_dot(y1, w2a_ref[...]) + _dot(y0, w2b_ref[...])
                     + _rep16(c2_ref[0]))
    return _lane_max16(y0), _lane_max16(y1), _lane_max16(y2)


def _edge_body_t(ag_ref, c0_ref, c1_ref, c2_ref, rest_ref,
                 w1_ref, w2a_ref, w2b_ref,
                 wca2_ref, wca1_ref, wca0_ref, wcb_ref, bc_ref,
                 m2_ref, m1_ref, m0_ref, t_ref):
    m0, m1, m2 = _edge_stages(ag_ref, c0_ref, c1_ref, c2_ref,
                              w1_ref, w2a_ref, w2b_ref)
    m0_ref[0], m1_ref[0], m2_ref[0] = m0, m1, m2
    t_ref[0] = jax.nn.relu(
        _dot(m2, wca2_ref[...]) + _dot(m1, wca1_ref[...])
        + _dot(m0, wca0_ref[...]) + _dot(rest_ref[0], wcb_ref[...])
        + bc_ref[...])


def _edge_body(ag_ref, c0_ref, c1_ref, c2_ref,
               w1_ref, w2a_ref, w2b_ref,
               m2_ref, m1_ref, m0_ref):
    m0, m1, m2 = _edge_stages(ag_ref, c0_ref, c1_ref, c2_ref,
                              w1_ref, w2a_ref, w2b_ref)
    m0_ref[0], m1_ref[0], m2_ref[0] = m0, m1, m2


def _edge_call(ag, c0, c1, c2, w1, w2a, w2b, rest=None, wca=None, wcb=None,
               bc=None):
    cspec = pl.BlockSpec((1, M, CP), lambda b, i: (b, i, 0))
    wspec = pl.BlockSpec((LANES, LANES), lambda b, i: (0, 0))
    in_specs = [
        pl.BlockSpec((1, M, K * TW), lambda b, i: (b, i, 0)),
        cspec, cspec, cspec,
    ]
    mspec = pl.BlockSpec((1, M, CP), lambda b, i: (b, i, 0))
    mshape = jax.ShapeDtypeStruct((B, N, CP), jnp.float32)
    if rest is None:
        return pl.pallas_call(
            _edge_body,
            grid=(B, N // M),
            in_specs=in_specs + [wspec, wspec, wspec],
            out_specs=[mspec, mspec, mspec],
            out_shape=[mshape, mshape, mshape],
            compiler_params=_TC_PARAMS,
        )(ag, c0, c1, c2, w1, w2a, w2b)
    R = rest.shape[-1]
    wcaspec = pl.BlockSpec((CP, 48), lambda b, i: (0, 0))
    return pl.pallas_call(
        _edge_body_t,
        grid=(B, N // M),
        in_specs=in_specs + [
            pl.BlockSpec((1, M, R), lambda b, i: (b, i, 0)),
            wspec, wspec, wspec,
            wcaspec, wcaspec, wcaspec,
            pl.BlockSpec((R, 48), lambda b, i: (0, 0)),
            pl.BlockSpec((1, 48), lambda b, i: (0, 0)),
        ],
        out_specs=[mspec, mspec, mspec,
                   pl.BlockSpec((1, M, 48), lambda b, i: (b, i, 0))],
        out_shape=[mshape, mshape, mshape,
                   jax.ShapeDtypeStruct((B, N, 48), jnp.float32)],
        compiler_params=_TC_PARAMS,
    )(ag, c0, c1, c2, rest, w1, w2a, w2b, wca[0], wca[1], wca[2],
      wcb, bc.reshape(1, 48))


def _block(f, l_prev, W0, b0, W1, b1, W2, b2, Wc=None, bc=None):
    C = f.shape[-1]
    w0a = _pad_rc(W0[:C], C, TW)
    cw0 = _pad_rc(W0[C:] - W0[:C], C, CP)
    cw1 = _pad_rc(W1[24:], C, CP)
    cw2 = _pad_rc(W2[48:], C, CP)
    b0p = _pad_rc(b0.reshape(1, 24), 1, CP)
    b1p = _pad_rc(b1.reshape(1, 24), 1, CP)
    b2p = _pad_rc(b2.reshape(1, 24), 1, CP)

    ft = jnp.swapaxes(f, 1, 2)
    gidx, A, c0, c1, c2 = _knn_call(f, ft, w0a, cw0, cw1, cw2, b0p, b1p, b2p)

    ag = _sc_gather(A.reshape(B * N, TW), gidx.reshape(1, _NIDX))
    ag = ag.reshape(B, N, K * TW)

    eye = jnp.eye(K, dtype=jnp.float32)
    w1bd = jnp.kron(eye, _pad_rc(W1[:24], CP, CP))
    w2abd = jnp.kron(eye, _pad_rc(W2[:24], CP, CP))
    w2bbd = jnp.kron(eye, _pad_rc(W2[24:48], CP, CP))

    if Wc is None:
        m2, m1, m0 = _edge_call(ag, c0, c1, c2, w1bd, w2abd, w2bbd)
        t = None
    else:
        rest = jnp.concatenate([f, l_prev], axis=-1)
        wca = [_pad_rc(Wc[0:24], CP, 48), _pad_rc(Wc[24:48], CP, 48),
               _pad_rc(Wc[48:72], CP, 48)]
        wcb = Wc[72:]
        m2, m1, m0, t = _edge_call(ag, c0, c1, c2, w1bd, w2abd, w2bbd,
                                   rest=rest, wca=wca, wcb=wcb, bc=bc)
    maxes = jnp.concatenate([m2[..., :24], m1[..., :24], m0[..., :24]],
                            axis=-1)
    return maxes, t


def kernel(pc, W_in, b_in, Wd0_0, bd0_0, Wd0_1, bd0_1, Wd0_2, bd0_2,
           Wc1, bc1, Wd1_0, bd1_0, Wd1_1, bd1_1, Wd1_2, bd1_2,
           Wc2, bc2, Wd2_0, bd2_0, Wd2_1, bd2_1, Wd2_2, bd2_2,
           Wc3, bc3, Wd3_0, bd3_0, Wd3_1, bd3_1, Wd3_2, bd3_2):
    l0 = _l0_call(pc, W_in, b_in)

    mx1, t1 = _block(l0, l0, Wd0_0, bd0_0, Wd0_1, bd0_1, Wd0_2, bd0_2,
                     Wc=Wc1, bc=bc1)
    l1 = jnp.concatenate([mx1, l0, l0], axis=-1)

    mx2, t2 = _block(t1, l1, Wd1_0, bd1_0, Wd1_1, bd1_1, Wd1_2, bd1_2,
                     Wc=Wc2, bc=bc2)
    l2 = jnp.concatenate([mx2, t1, l1], axis=-1)

    mx3, t3 = _block(t2, l2, Wd2_0, bd2_0, Wd2_1, bd2_1, Wd2_2, bd2_2,
                     Wc=Wc3, bc=bc3)
    l3 = jnp.concatenate([mx3, t2, l2], axis=-1)

    mx4, _ = _block(t3, l3, Wd3_0, bd3_0, Wd3_1, bd3_1, Wd3_2, bd3_2)
    return jnp.concatenate([mx4, t3, l3], axis=-1)


# speedup vs baseline: 12.9939x; 1.1984x over previous
import functools

import jax
import jax.numpy as jnp
import numpy as np
from jax.experimental import pallas as pl
from jax.experimental.pallas import tpu as pltpu
from jax.experimental.pallas import tpu_sc as plsc

N = 2048
B = 4
K = 16
M = 256
CP = 32
TW = 128
LANES = K * CP


_TC_PARAMS = pltpu.CompilerParams(
    dimension_semantics=("parallel", "arbitrary"))


def _dot(a, b):
    return jax.lax.dot_general(a, b, (((1,), (0,)), ((), ())),
                               preferred_element_type=jnp.float32)


def _pad_rc(w, rows, cols):
    return jnp.pad(w, ((0, rows - w.shape[0]), (0, cols - w.shape[1])))


def _l0_body(pc_ref, w_ref, b_ref, o_ref):
    o_ref[0] = jax.nn.relu(_dot(pc_ref[0], w_ref[...]) + b_ref[...])


def _l0_call(pc, W_in, b_in):
    return pl.pallas_call(
        _l0_body,
        grid=(B,),
        in_specs=[
            pl.BlockSpec((1, N, 3), lambda b: (b, 0, 0)),
            pl.BlockSpec((3, 24), lambda b: (0, 0)),
            pl.BlockSpec((1, 24), lambda b: (0, 0)),
        ],
        out_specs=pl.BlockSpec((1, N, 24), lambda b: (b, 0, 0)),
        out_shape=jax.ShapeDtypeStruct((B, N, 24), jnp.float32),
        compiler_params=pltpu.CompilerParams(
            dimension_semantics=("parallel",)),
    )(pc, W_in, b_in.reshape(1, 24))


def _knn_body(f_ref, ft_ref, w0a_ref, cw0_ref, cw1_ref, cw2_ref,
              b0_ref, b1_ref, b2_ref,
              gidx_ref, a_ref, c0_ref, c1_ref, c2_ref):
    b = pl.program_id(0)
    f = f_ref[0]
    ft = ft_ref[0]
    dot = _dot(f, ft)
    sq_all = jnp.sum(ft * ft, axis=0, keepdims=True)
    sq_t = jnp.sum(f * f, axis=1, keepdims=True)
    D = sq_t - 2.0 * dot + sq_all

    ng = N // 128
    dbits = jax.lax.bitcast_convert_type(jnp.maximum(D, 0.0) + 1.0,
                                         jnp.int32)
    dp = [jax.lax.bitcast_convert_type(
              (dbits[:, g * 128:(g + 1) * 128] & jnp.int32(-16)) | g,
              jnp.float32)
          for g in range(ng)]
    val = dp[0]
    for g in range(1, ng):
        val = jnp.minimum(val, dp[g])
    lane_iota = jax.lax.broadcasted_iota(
        jnp.int32, (M, 128), 1).astype(jnp.float32)
    bigf = jnp.float32(3e38)
    inf = jnp.float32(np.inf)
    cols = []
    for _ in range(K):
        mnp = jnp.min(val, axis=1, keepdims=True)
        lane = jnp.min(jnp.where(val == mnp, lane_iota, bigf),
                       axis=1, keepdims=True).astype(jnp.int32)
        grp = jax.lax.bitcast_convert_type(mnp, jnp.int32) & 15
        cols.append(grp * 128 + lane)
        new_dp = []
        val = None
        for g in range(ng):
            dg = jnp.where(dp[g] == mnp, inf, dp[g])
            new_dp.append(dg)
            val = dg if val is None else jnp.minimum(val, dg)
        dp = new_dp
    idx = jnp.concatenate(cols, axis=1)
    gidx_ref[0] = idx + b * N

    a_ref[0] = _dot(f, w0a_ref[...])
    c0_ref[0] = _dot(f, cw0_ref[...]) + b0_ref[...]
    c1_ref[0] = _dot(f, cw1_ref[...]) + b1_ref[...]
    c2_ref[0] = _dot(f, cw2_ref[...]) + b2_ref[...]


def _knn_call(f, ft, w0a, cw0, cw1, cw2, b0, b1, b2):
    C = f.shape[-1]
    wspec = pl.BlockSpec((C, CP), lambda b, i: (0, 0))
    bspec = pl.BlockSpec((1, CP), lambda b, i: (0, 0))
    ospec = pl.BlockSpec((1, M, CP), lambda b, i: (b, i, 0))
    return pl.pallas_call(
        _knn_body,
        grid=(B, N // M),
        in_specs=[
            pl.BlockSpec((1, M, C), lambda b, i: (b, i, 0)),
            pl.BlockSpec((1, C, N), lambda b, i: (b, 0, 0)),
            pl.BlockSpec((C, TW), lambda b, i: (0, 0)),
            wspec, wspec, wspec, bspec, bspec, bspec,
        ],
        out_specs=[
            pl.BlockSpec((1, M, K), lambda b, i: (b, i, 0)),
            pl.BlockSpec((1, M, TW), lambda b, i: (b, i, 0)),
            ospec, ospec, ospec,
        ],
        out_shape=[
            jax.ShapeDtypeStruct((B, N, K), jnp.int32),
            jax.ShapeDtypeStruct((B, N, TW), jnp.float32),
            jax.ShapeDtypeStruct((B, N, CP), jnp.float32),
            jax.ShapeDtypeStruct((B, N, CP), jnp.float32),
            jax.ShapeDtypeStruct((B, N, CP), jnp.float32),
        ],
        compiler_params=_TC_PARAMS,
    )(f, ft, w0a, cw0, cw1, cw2, b0, b1, b2)


_GW = 128
_NIDX = B * N * K


def _sc_gather(table, indices):
    mesh = plsc.VectorSubcoreMesh(core_axis_name="core",
                                  subcore_axis_name="subcore")

    @functools.partial(
        pl.kernel,
        out_type=jax.ShapeDtypeStruct((_NIDX, TW), jnp.float32),
        mesh=mesh,
    )
    def gather_kernel(x_hbm, i_hbm, o_hbm):
        def body(i_vmem, o_vmem):
            pltpu.sync_copy(x_hbm.at[i_vmem.at[0]], o_vmem)

        pltpu.emit_pipeline(
            body,
            grid=(_NIDX // _GW,),
            in_specs=[pl.BlockSpec((1, _GW), index_map=lambda i: (0, i))],
            out_specs=[pl.BlockSpec((_GW, TW), index_map=lambda i: (i, 0))],
            core_axis_name=("core", "subcore"),
            dimension_semantics=(pltpu.PARALLEL,),
        )(i_hbm, o_hbm)

    return gather_kernel(table, indices)


def _rep16(c):
    return jnp.concatenate([c] * K, axis=1)


def _lane_max16(y):
    a = jnp.maximum(y[:, :256], y[:, 256:])
    a = jnp.maximum(a[:, :128], a[:, 128:])
    a = jnp.maximum(a[:, :64], a[:, 64:])
    return jnp.maximum(a[:, :32], a[:, 32:])


def _edge_stages(ag_ref, c0_ref, c1_ref, c2_ref, w1_ref, w2a_ref, w2b_ref):
    agw = ag_ref[0]
    ag = jnp.concatenate([agw[:, j * TW:j * TW + CP] for j in range(K)],
                         axis=1)
    y0 = jax.nn.relu(ag + _rep16(c0_ref[0]))
    y1 = jax.nn.relu(_dot(y0, w1_ref[...]) + _rep16(c1_ref[0]))
    y2 = jax.nn.relu(_dot(y1, w2a_ref[...]) + _dot(y0, w2b_ref[...])
                     + _rep16(c2_ref[0]))
    return _lane_max16(y0), _lane_max16(y1), _lane_max16(y2)


def _edge_body_t(ag_ref, c0_ref, c1_ref, c2_ref, rest_ref,
                 w1_ref, w2a_ref, w2b_ref,
                 wca2_ref, wca1_ref, wca0_ref, wcb_ref, bc_ref,
                 m2_ref, m1_ref, m0_ref, t_ref):
    m0, m1, m2 = _edge_stages(ag_ref, c0_ref, c1_ref, c2_ref,
                              w1_ref, w2a_ref, w2b_ref)
    m0_ref[0], m1_ref[0], m2_ref[0] = m0, m1, m2
    t_ref[0] = jax.nn.relu(
        _dot(m2, wca2_ref[...]) + _dot(m1, wca1_ref[...])
        + _dot(m0, wca0_ref[...]) + _dot(rest_ref[0], wcb_ref[...])
        + bc_ref[...])


def _edge_body(ag_ref, c0_ref, c1_ref, c2_ref,
               w1_ref, w2a_ref, w2b_ref,
               m2_ref, m1_ref, m0_ref):
    m0, m1, m2 = _edge_stages(ag_ref, c0_ref, c1_ref, c2_ref,
                              w1_ref, w2a_ref, w2b_ref)
    m0_ref[0], m1_ref[0], m2_ref[0] = m0, m1, m2


def _edge_call(ag, c0, c1, c2, w1, w2a, w2b, rest=None, wca=None, wcb=None,
               bc=None):
    cspec = pl.BlockSpec((1, M, CP), lambda b, i: (b, i, 0))
    wspec = pl.BlockSpec((LANES, LANES), lambda b, i: (0, 0))
    in_specs = [
        pl.BlockSpec((1, M, K * TW), lambda b, i: (b, i, 0)),
        cspec, cspec, cspec,
    ]
    mspec = pl.BlockSpec((1, M, CP), lambda b, i: (b, i, 0))
    mshape = jax.ShapeDtypeStruct((B, N, CP), jnp.float32)
    if rest is None:
        return pl.pallas_call(
            _edge_body,
            grid=(B, N // M),
            in_specs=in_specs + [wspec, wspec, wspec],
            out_specs=[mspec, mspec, mspec],
            out_shape=[mshape, mshape, mshape],
            compiler_params=_TC_PARAMS,
        )(ag, c0, c1, c2, w1, w2a, w2b)
    R = rest.shape[-1]
    wcaspec = pl.BlockSpec((CP, 48), lambda b, i: (0, 0))
    return pl.pallas_call(
        _edge_body_t,
        grid=(B, N // M),
        in_specs=in_specs + [
            pl.BlockSpec((1, M, R), lambda b, i: (b, i, 0)),
            wspec, wspec, wspec,
            wcaspec, wcaspec, wcaspec,
            pl.BlockSpec((R, 48), lambda b, i: (0, 0)),
            pl.BlockSpec((1, 48), lambda b, i: (0, 0)),
        ],
        out_specs=[mspec, mspec, mspec,
                   pl.BlockSpec((1, M, 48), lambda b, i: (b, i, 0))],
        out_shape=[mshape, mshape, mshape,
                   jax.ShapeDtypeStruct((B, N, 48), jnp.float32)],
        compiler_params=_TC_PARAMS,
    )(ag, c0, c1, c2, rest, w1, w2a, w2b, wca[0], wca[1], wca[2],
      wcb, bc.reshape(1, 48))


def _block(f, l_prev, W0, b0, W1, b1, W2, b2, Wc=None, bc=None):
    C = f.shape[-1]
    w0a = _pad_rc(W0[:C], C, TW)
    cw0 = _pad_rc(W0[C:] - W0[:C], C, CP)
    cw1 = _pad_rc(W1[24:], C, CP)
    cw2 = _pad_rc(W2[48:], C, CP)
    b0p = _pad_rc(b0.reshape(1, 24), 1, CP)
    b1p = _pad_rc(b1.reshape(1, 24), 1, CP)
    b2p = _pad_rc(b2.reshape(1, 24), 1, CP)

    ft = jnp.swapaxes(f, 1, 2)
    gidx, A, c0, c1, c2 = _knn_call(f, ft, w0a, cw0, cw1, cw2, b0p, b1p, b2p)

    ag = _sc_gather(A.reshape(B * N, TW), gidx.reshape(1, _NIDX))
    ag = ag.reshape(B, N, K * TW)

    eye = jnp.eye(K, dtype=jnp.float32)
    w1bd = jnp.kron(eye, _pad_rc(W1[:24], CP, CP))
    w2abd = jnp.kron(eye, _pad_rc(W2[:24], CP, CP))
    w2bbd = jnp.kron(eye, _pad_rc(W2[24:48], CP, CP))

    if Wc is None:
        m2, m1, m0 = _edge_call(ag, c0, c1, c2, w1bd, w2abd, w2bbd)
        t = None
    else:
        rest = jnp.concatenate([f, l_prev], axis=-1)
        wca = [_pad_rc(Wc[0:24], CP, 48), _pad_rc(Wc[24:48], CP, 48),
               _pad_rc(Wc[48:72], CP, 48)]
        wcb = Wc[72:]
        m2, m1, m0, t = _edge_call(ag, c0, c1, c2, w1bd, w2abd, w2bbd,
                                   rest=rest, wca=wca, wcb=wcb, bc=bc)
    maxes = jnp.concatenate([m2[..., :24], m1[..., :24], m0[..., :24]],
                            axis=-1)
    return maxes, t


def kernel(pc, W_in, b_in, Wd0_0, bd0_0, Wd0_1, bd0_1, Wd0_2, bd0_2,
           Wc1, bc1, Wd1_0, bd1_0, Wd1_1, bd1_1, Wd1_2, bd1_2,
           Wc2, bc2, Wd2_0, bd2_0, Wd2_1, bd2_1, Wd2_2, bd2_2,
           Wc3, bc3, Wd3_0, bd3_0, Wd3_1, bd3_1, Wd3_2, bd3_2):
    l0 = _l0_call(pc, W_in, b_in)

    mx1, t1 = _block(l0, l0, Wd0_0, bd0_0, Wd0_1, bd0_1, Wd0_2, bd0_2,
                     Wc=Wc1, bc=bc1)
    l1 = jnp.concatenate([mx1, l0, l0], axis=-1)

    mx2, t2 = _block(t1, l1, Wd1_0, bd1_0, Wd1_1, bd1_1, Wd1_2, bd1_2,
                     Wc=Wc2, bc=bc2)
    l2 = jnp.concatenate([mx2, t1, l1], axis=-1)

    mx3, t3 = _block(t2, l2, Wd2_0, bd2_0, Wd2_1, bd2_1, Wd2_2, bd2_2,
                     Wc=Wc3, bc=bc3)
    l3 = jnp.concatenate([mx3, t2, l2], axis=-1)

    mx4, _ = _block(t3, l3, Wd3_0, bd3_0, Wd3_1, bd3_1, Wd3_2, bd3_2)
    return jnp.concatenate([mx4, t3, l3], axis=-1)


# speedup vs baseline: 13.1247x; 1.0101x over previous
import functools

import jax
import jax.numpy as jnp
import numpy as np
from jax.experimental import pallas as pl
from jax.experimental.pallas import tpu as pltpu
from jax.experimental.pallas import tpu_sc as plsc

N = 2048
B = 4
K = 16
M = 256
CP = 32
TW = 128
LANES = K * CP


_TC_PARAMS = pltpu.CompilerParams(
    dimension_semantics=("parallel", "arbitrary"))


def _dot(a, b):
    return jax.lax.dot_general(a, b, (((1,), (0,)), ((), ())),
                               preferred_element_type=jnp.float32)


def _pad_rc(w, rows, cols):
    return jnp.pad(w, ((0, rows - w.shape[0]), (0, cols - w.shape[1])))


def _l0_body(pc_ref, w_ref, b_ref, o_ref):
    o_ref[0] = jax.nn.relu(_dot(pc_ref[0], w_ref[...]) + b_ref[...])


def _l0_call(pc, W_in, b_in):
    return pl.pallas_call(
        _l0_body,
        grid=(B,),
        in_specs=[
            pl.BlockSpec((1, N, 3), lambda b: (b, 0, 0)),
            pl.BlockSpec((3, 24), lambda b: (0, 0)),
            pl.BlockSpec((1, 24), lambda b: (0, 0)),
        ],
        out_specs=pl.BlockSpec((1, N, 24), lambda b: (b, 0, 0)),
        out_shape=jax.ShapeDtypeStruct((B, N, 24), jnp.float32),
        compiler_params=pltpu.CompilerParams(
            dimension_semantics=("parallel",)),
    )(pc, W_in, b_in.reshape(1, 24))


def _knn_body(f_ref, ft_ref, w0a_ref, cw0_ref, cw1_ref, cw2_ref,
              b0_ref, b1_ref, b2_ref,
              gidx_ref, a_ref, c0_ref, c1_ref, c2_ref):
    b = pl.program_id(0)
    f = f_ref[0]
    ft = ft_ref[0]
    dot = _dot(f, ft)
    sq_all = jnp.sum(ft * ft, axis=0, keepdims=True)
    sq_t = jnp.sum(f * f, axis=1, keepdims=True)
    D = sq_t - 2.0 * dot + sq_all

    ng = N // 128
    dbits = jax.lax.bitcast_convert_type(jnp.maximum(D, 0.0) + 1.0,
                                         jnp.int32)
    dp = [jax.lax.bitcast_convert_type(
              (dbits[:, g * 128:(g + 1) * 128] & jnp.int32(-16)) | g,
              jnp.float32)
          for g in range(ng)]
    val = dp[0]
    for g in range(1, ng):
        val = jnp.minimum(val, dp[g])
    lane_iota = jax.lax.broadcasted_iota(
        jnp.int32, (M, 128), 1).astype(jnp.float32)
    bigf = jnp.float32(3e38)
    inf = jnp.float32(np.inf)
    cols = []
    for _ in range(K):
        mnp = jnp.min(val, axis=1, keepdims=True)
        lane = jnp.min(jnp.where(val == mnp, lane_iota, bigf),
                       axis=1, keepdims=True).astype(jnp.int32)
        grp = jax.lax.bitcast_convert_type(mnp, jnp.int32) & 15
        cols.append(grp * 128 + lane)
        new_dp = []
        val = None
        for g in range(ng):
            dg = jnp.where(dp[g] == mnp, inf, dp[g])
            new_dp.append(dg)
            val = dg if val is None else jnp.minimum(val, dg)
        dp = new_dp
    idx = jnp.concatenate(cols, axis=1)
    gidx_ref[0] = idx + b * N

    a_ref[0] = _dot(f, w0a_ref[...])
    c0_ref[0] = _dot(f, cw0_ref[...]) + b0_ref[...]
    c1_ref[0] = _dot(f, cw1_ref[...]) + b1_ref[...]
    c2_ref[0] = _dot(f, cw2_ref[...]) + b2_ref[...]


def _knn_call(f, ft, w0a, cw0, cw1, cw2, b0, b1, b2):
    C = f.shape[-1]
    wspec = pl.BlockSpec((C, CP), lambda b, i: (0, 0))
    bspec = pl.BlockSpec((1, CP), lambda b, i: (0, 0))
    ospec = pl.BlockSpec((1, M, CP), lambda b, i: (b, i, 0))
    return pl.pallas_call(
        _knn_body,
        grid=(B, N // M),
        in_specs=[
            pl.BlockSpec((1, M, C), lambda b, i: (b, i, 0)),
            pl.BlockSpec((1, C, N), lambda b, i: (b, 0, 0)),
            pl.BlockSpec((C, TW), lambda b, i: (0, 0)),
            wspec, wspec, wspec, bspec, bspec, bspec,
        ],
        out_specs=[
            pl.BlockSpec((1, M, K), lambda b, i: (b, i, 0)),
            pl.BlockSpec((1, M, TW), lambda b, i: (b, i, 0)),
            ospec, ospec, ospec,
        ],
        out_shape=[
            jax.ShapeDtypeStruct((B, N, K), jnp.int32),
            jax.ShapeDtypeStruct((B, N, TW), jnp.float32),
            jax.ShapeDtypeStruct((B, N, CP), jnp.float32),
            jax.ShapeDtypeStruct((B, N, CP), jnp.float32),
            jax.ShapeDtypeStruct((B, N, CP), jnp.float32),
        ],
        compiler_params=_TC_PARAMS,
    )(f, ft, w0a, cw0, cw1, cw2, b0, b1, b2)


_GW = 256
_NIDX = B * N * K


def _sc_gather(table, indices):
    mesh = plsc.VectorSubcoreMesh(core_axis_name="core",
                                  subcore_axis_name="subcore")

    @functools.partial(
        pl.kernel,
        out_type=jax.ShapeDtypeStruct((_NIDX, TW), jnp.float32),
        mesh=mesh,
    )
    def gather_kernel(x_hbm, i_hbm, o_hbm):
        def body(i_vmem, o_vmem):
            pltpu.sync_copy(x_hbm.at[i_vmem.at[0]], o_vmem)

        pltpu.emit_pipeline(
            body,
            grid=(_NIDX // _GW,),
            in_specs=[pl.BlockSpec((1, _GW), index_map=lambda i: (0, i))],
            out_specs=[pl.BlockSpec((_GW, TW), index_map=lambda i: (i, 0))],
            core_axis_name=("core", "subcore"),
            dimension_semantics=(pltpu.PARALLEL,),
        )(i_hbm, o_hbm)

    return gather_kernel(table, indices)


def _rep16(c):
    return jnp.concatenate([c] * K, axis=1)


def _lane_max16(y):
    a = jnp.maximum(y[:, :256], y[:, 256:])
    a = jnp.maximum(a[:, :128], a[:, 128:])
    a = jnp.maximum(a[:, :64], a[:, 64:])
    return jnp.maximum(a[:, :32], a[:, 32:])


def _edge_stages(ag_ref, c0_ref, c1_ref, c2_ref, w1_ref, w2a_ref, w2b_ref):
    agw = ag_ref[0]
    ag = jnp.concatenate([agw[:, j * TW:j * TW + CP] for j in range(K)],
                         axis=1)
    y0 = jax.nn.relu(ag + _rep16(c0_ref[0]))
    y1 = jax.nn.relu(_dot(y0, w1_ref[...]) + _rep16(c1_ref[0]))
    y2 = jax.nn.relu(_dot(y1, w2a_ref[...]) + _dot(y0, w2b_ref[...])
                     + _rep16(c2_ref[0]))
    return _lane_max16(y0), _lane_max16(y1), _lane_max16(y2)


def _edge_body_t(ag_ref, c0_ref, c1_ref, c2_ref, rest_ref,
                 w1_ref, w2a_ref, w2b_ref,
                 wca2_ref, wca1_ref, wca0_ref, wcb_ref, bc_ref,
                 m2_ref, m1_ref, m0_ref, t_ref):
    m0, m1, m2 = _edge_stages(ag_ref, c0_ref, c1_ref, c2_ref,
                              w1_ref, w2a_ref, w2b_ref)
    m0_ref[0], m1_ref[0], m2_ref[0] = m0, m1, m2
    t_ref[0] = jax.nn.relu(
        _dot(m2, wca2_ref[...]) + _dot(m1, wca1_ref[...])
        + _dot(m0, wca0_ref[...]) + _dot(rest_ref[0], wcb_ref[...])
        + bc_ref[...])


def _edge_body(ag_ref, c0_ref, c1_ref, c2_ref,
               w1_ref, w2a_ref, w2b_ref,
               m2_ref, m1_ref, m0_ref):
    m0, m1, m2 = _edge_stages(ag_ref, c0_ref, c1_ref, c2_ref,
                              w1_ref, w2a_ref, w2b_ref)
    m0_ref[0], m1_ref[0], m2_ref[0] = m0, m1, m2


def _edge_call(ag, c0, c1, c2, w1, w2a, w2b, rest=None, wca=None, wcb=None,
               bc=None):
    cspec = pl.BlockSpec((1, M, CP), lambda b, i: (b, i, 0))
    wspec = pl.BlockSpec((LANES, LANES), lambda b, i: (0, 0))
    in_specs = [
        pl.BlockSpec((1, M, K * TW), lambda b, i: (b, i, 0)),
        cspec, cspec, cspec,
    ]
    mspec = pl.BlockSpec((1, M, CP), lambda b, i: (b, i, 0))
    mshape = jax.ShapeDtypeStruct((B, N, CP), jnp.float32)
    if rest is None:
        return pl.pallas_call(
            _edge_body,
            grid=(B, N // M),
            in_specs=in_specs + [wspec, wspec, wspec],
            out_specs=[mspec, mspec, mspec],
            out_shape=[mshape, mshape, mshape],
            compiler_params=_TC_PARAMS,
        )(ag, c0, c1, c2, w1, w2a, w2b)
    R = rest.shape[-1]
    wcaspec = pl.BlockSpec((CP, 48), lambda b, i: (0, 0))
    return pl.pallas_call(
        _edge_body_t,
        grid=(B, N // M),
        in_specs=in_specs + [
            pl.BlockSpec((1, M, R), lambda b, i: (b, i, 0)),
            wspec, wspec, wspec,
            wcaspec, wcaspec, wcaspec,
            pl.BlockSpec((R, 48), lambda b, i: (0, 0)),
            pl.BlockSpec((1, 48), lambda b, i: (0, 0)),
        ],
        out_specs=[mspec, mspec, mspec,
                   pl.BlockSpec((1, M, 48), lambda b, i: (b, i, 0))],
        out_shape=[mshape, mshape, mshape,
                   jax.ShapeDtypeStruct((B, N, 48), jnp.float32)],
        compiler_params=_TC_PARAMS,
    )(ag, c0, c1, c2, rest, w1, w2a, w2b, wca[0], wca[1], wca[2],
      wcb, bc.reshape(1, 48))


def _block(f, l_prev, W0, b0, W1, b1, W2, b2, Wc=None, bc=None):
    C = f.shape[-1]
    w0a = _pad_rc(W0[:C], C, TW)
    cw0 = _pad_rc(W0[C:] - W0[:C], C, CP)
    cw1 = _pad_rc(W1[24:], C, CP)
    cw2 = _pad_rc(W2[48:], C, CP)
    b0p = _pad_rc(b0.reshape(1, 24), 1, CP)
    b1p = _pad_rc(b1.reshape(1, 24), 1, CP)
    b2p = _pad_rc(b2.reshape(1, 24), 1, CP)

    ft = jnp.swapaxes(f, 1, 2)
    gidx, A, c0, c1, c2 = _knn_call(f, ft, w0a, cw0, cw1, cw2, b0p, b1p, b2p)

    ag = _sc_gather(A.reshape(B * N, TW), gidx.reshape(1, _NIDX))
    ag = ag.reshape(B, N, K * TW)

    eye = jnp.eye(K, dtype=jnp.float32)
    w1bd = jnp.kron(eye, _pad_rc(W1[:24], CP, CP))
    w2abd = jnp.kron(eye, _pad_rc(W2[:24], CP, CP))
    w2bbd = jnp.kron(eye, _pad_rc(W2[24:48], CP, CP))

    if Wc is None:
        m2, m1, m0 = _edge_call(ag, c0, c1, c2, w1bd, w2abd, w2bbd)
        t = None
    else:
        rest = jnp.concatenate([f, l_prev], axis=-1)
        wca = [_pad_rc(Wc[0:24], CP, 48), _pad_rc(Wc[24:48], CP, 48),
               _pad_rc(Wc[48:72], CP, 48)]
        wcb = Wc[72:]
        m2, m1, m0, t = _edge_call(ag, c0, c1, c2, w1bd, w2abd, w2bbd,
                                   rest=rest, wca=wca, wcb=wcb, bc=bc)
    maxes = jnp.concatenate([m2[..., :24], m1[..., :24], m0[..., :24]],
                            axis=-1)
    return maxes, t


def kernel(pc, W_in, b_in, Wd0_0, bd0_0, Wd0_1, bd0_1, Wd0_2, bd0_2,
           Wc1, bc1, Wd1_0, bd1_0, Wd1_1, bd1_1, Wd1_2, bd1_2,
           Wc2, bc2, Wd2_0, bd2_0, Wd2_1, bd2_1, Wd2_2, bd2_2,
           Wc3, bc3, Wd3_0, bd3_0, Wd3_1, bd3_1, Wd3_2, bd3_2):
    l0 = _l0_call(pc, W_in, b_in)

    mx1, t1 = _block(l0, l0, Wd0_0, bd0_0, Wd0_1, bd0_1, Wd0_2, bd0_2,
                     Wc=Wc1, bc=bc1)
    l1 = jnp.concatenate([mx1, l0, l0], axis=-1)

    mx2, t2 = _block(t1, l1, Wd1_0, bd1_0, Wd1_1, bd1_1, Wd1_2, bd1_2,
                     Wc=Wc2, bc=bc2)
    l2 = jnp.concatenate([mx2, t1, l1], axis=-1)

    mx3, t3 = _block(t2, l2, Wd2_0, bd2_0, Wd2_1, bd2_1, Wd2_2, bd2_2,
                     Wc=Wc3, bc=bc3)
    l3 = jnp.concatenate([mx3, t2, l2], axis=-1)

    mx4, _ = _block(t3, l3, Wd3_0, bd3_0, Wd3_1, bd3_1, Wd3_2, bd3_2)
    return jnp.concatenate([mx4, t3, l3], axis=-1)


# speedup vs baseline: 13.6212x; 1.0378x over previous
import functools

import jax
import jax.numpy as jnp
import numpy as np
from jax.experimental import pallas as pl
from jax.experimental.pallas import tpu as pltpu
from jax.experimental.pallas import tpu_sc as plsc

N = 2048
B = 4
K = 16
M = 512
CP = 32
TW = 128
LANES = K * CP


_TC_PARAMS = pltpu.CompilerParams(
    dimension_semantics=("parallel", "arbitrary"))


def _dot(a, b):
    return jax.lax.dot_general(a, b, (((1,), (0,)), ((), ())),
                               preferred_element_type=jnp.float32)


def _pad_rc(w, rows, cols):
    return jnp.pad(w, ((0, rows - w.shape[0]), (0, cols - w.shape[1])))


def _l0_body(pc_ref, w_ref, b_ref, o_ref):
    o_ref[0] = jax.nn.relu(_dot(pc_ref[0], w_ref[...]) + b_ref[...])


def _l0_call(pc, W_in, b_in):
    return pl.pallas_call(
        _l0_body,
        grid=(B,),
        in_specs=[
            pl.BlockSpec((1, N, 3), lambda b: (b, 0, 0)),
            pl.BlockSpec((3, 24), lambda b: (0, 0)),
            pl.BlockSpec((1, 24), lambda b: (0, 0)),
        ],
        out_specs=pl.BlockSpec((1, N, 24), lambda b: (b, 0, 0)),
        out_shape=jax.ShapeDtypeStruct((B, N, 24), jnp.float32),
        compiler_params=pltpu.CompilerParams(
            dimension_semantics=("parallel",)),
    )(pc, W_in, b_in.reshape(1, 24))


def _knn_body(f_ref, ft_ref, w0a_ref, cw0_ref, cw1_ref, cw2_ref,
              b0_ref, b1_ref, b2_ref,
              gidx_ref, a_ref, c0_ref, c1_ref, c2_ref):
    b = pl.program_id(0)
    f = f_ref[0]
    ft = ft_ref[0]
    dot = _dot(f, ft)
    sq_all = jnp.sum(ft * ft, axis=0, keepdims=True)
    sq_t = jnp.sum(f * f, axis=1, keepdims=True)
    D = sq_t - 2.0 * dot + sq_all

    ng = N // 128
    dbits = jax.lax.bitcast_convert_type(jnp.maximum(D, 0.0) + 1.0,
                                         jnp.int32)
    dp = [jax.lax.bitcast_convert_type(
              (dbits[:, g * 128:(g + 1) * 128] & jnp.int32(-16)) | g,
              jnp.float32)
          for g in range(ng)]
    val = dp[0]
    for g in range(1, ng):
        val = jnp.minimum(val, dp[g])
    lane_iota = jax.lax.broadcasted_iota(
        jnp.int32, (M, 128), 1).astype(jnp.float32)
    bigf = jnp.float32(3e38)
    inf = jnp.float32(np.inf)
    cols = []
    for _ in range(K):
        mnp = jnp.min(val, axis=1, keepdims=True)
        lane = jnp.min(jnp.where(val == mnp, lane_iota, bigf),
                       axis=1, keepdims=True).astype(jnp.int32)
        grp = jax.lax.bitcast_convert_type(mnp, jnp.int32) & 15
        cols.append(grp * 128 + lane)
        new_dp = []
        val = None
        for g in range(ng):
            dg = jnp.where(dp[g] == mnp, inf, dp[g])
            new_dp.append(dg)
            val = dg if val is None else jnp.minimum(val, dg)
        dp = new_dp
    idx = jnp.concatenate(cols, axis=1)
    gidx_ref[0] = idx + b * N

    a_ref[0] = _dot(f, w0a_ref[...])
    c0_ref[0] = _dot(f, cw0_ref[...]) + b0_ref[...]
    c1_ref[0] = _dot(f, cw1_ref[...]) + b1_ref[...]
    c2_ref[0] = _dot(f, cw2_ref[...]) + b2_ref[...]


def _knn_call(f, ft, w0a, cw0, cw1, cw2, b0, b1, b2):
    C = f.shape[-1]
    wspec = pl.BlockSpec((C, CP), lambda b, i: (0, 0))
    bspec = pl.BlockSpec((1, CP), lambda b, i: (0, 0))
    ospec = pl.BlockSpec((1, M, CP), lambda b, i: (b, i, 0))
    return pl.pallas_call(
        _knn_body,
        grid=(B, N // M),
        in_specs=[
            pl.BlockSpec((1, M, C), lambda b, i: (b, i, 0)),
            pl.BlockSpec((1, C, N), lambda b, i: (b, 0, 0)),
            pl.BlockSpec((C, TW), lambda b, i: (0, 0)),
            wspec, wspec, wspec, bspec, bspec, bspec,
        ],
        out_specs=[
            pl.BlockSpec((1, M, K), lambda b, i: (b, i, 0)),
            pl.BlockSpec((1, M, TW), lambda b, i: (b, i, 0)),
            ospec, ospec, ospec,
        ],
        out_shape=[
            jax.ShapeDtypeStruct((B, N, K), jnp.int32),
            jax.ShapeDtypeStruct((B, N, TW), jnp.float32),
            jax.ShapeDtypeStruct((B, N, CP), jnp.float32),
            jax.ShapeDtypeStruct((B, N, CP), jnp.float32),
            jax.ShapeDtypeStruct((B, N, CP), jnp.float32),
        ],
        compiler_params=_TC_PARAMS,
    )(f, ft, w0a, cw0, cw1, cw2, b0, b1, b2)


_GW = 256
_NIDX = B * N * K


def _sc_gather(table, indices):
    mesh = plsc.VectorSubcoreMesh(core_axis_name="core",
                                  subcore_axis_name="subcore")

    @functools.partial(
        pl.kernel,
        out_type=jax.ShapeDtypeStruct((_NIDX, TW), jnp.float32),
        mesh=mesh,
    )
    def gather_kernel(x_hbm, i_hbm, o_hbm):
        def body(i_vmem, o_vmem):
            pltpu.sync_copy(x_hbm.at[i_vmem.at[0]], o_vmem)

        pltpu.emit_pipeline(
            body,
            grid=(_NIDX // _GW,),
            in_specs=[pl.BlockSpec((1, _GW), index_map=lambda i: (0, i))],
            out_specs=[pl.BlockSpec((_GW, TW), index_map=lambda i: (i, 0))],
            core_axis_name=("core", "subcore"),
            dimension_semantics=(pltpu.PARALLEL,),
        )(i_hbm, o_hbm)

    return gather_kernel(table, indices)


def _rep16(c):
    return jnp.concatenate([c] * K, axis=1)


def _lane_max16(y):
    a = jnp.maximum(y[:, :256], y[:, 256:])
    a = jnp.maximum(a[:, :128], a[:, 128:])
    a = jnp.maximum(a[:, :64], a[:, 64:])
    return jnp.maximum(a[:, :32], a[:, 32:])


def _edge_stages(ag_ref, c0_ref, c1_ref, c2_ref, w1_ref, w2a_ref, w2b_ref):
    agw = ag_ref[0]
    ag = jnp.concatenate([agw[:, j * TW:j * TW + CP] for j in range(K)],
                         axis=1)
    y0 = jax.nn.relu(ag + _rep16(c0_ref[0]))
    y1 = jax.nn.relu(_dot(y0, w1_ref[...]) + _rep16(c1_ref[0]))
    y2 = jax.nn.relu(_dot(y1, w2a_ref[...]) + _dot(y0, w2b_ref[...])
                     + _rep16(c2_ref[0]))
    return _lane_max16(y0), _lane_max16(y1), _lane_max16(y2)


def _edge_body_t(ag_ref, c0_ref, c1_ref, c2_ref, rest_ref,
                 w1_ref, w2a_ref, w2b_ref,
                 wca2_ref, wca1_ref, wca0_ref, wcb_ref, bc_ref,
                 m2_ref, m1_ref, m0_ref, t_ref):
    m0, m1, m2 = _edge_stages(ag_ref, c0_ref, c1_ref, c2_ref,
                              w1_ref, w2a_ref, w2b_ref)
    m0_ref[0], m1_ref[0], m2_ref[0] = m0, m1, m2
    t_ref[0] = jax.nn.relu(
        _dot(m2, wca2_ref[...]) + _dot(m1, wca1_ref[...])
        + _dot(m0, wca0_ref[...]) + _dot(rest_ref[0], wcb_ref[...])
        + bc_ref[...])


def _edge_body(ag_ref, c0_ref, c1_ref, c2_ref,
               w1_ref, w2a_ref, w2b_ref,
               m2_ref, m1_ref, m0_ref):
    m0, m1, m2 = _edge_stages(ag_ref, c0_ref, c1_ref, c2_ref,
                              w1_ref, w2a_ref, w2b_ref)
    m0_ref[0], m1_ref[0], m2_ref[0] = m0, m1, m2


def _edge_call(ag, c0, c1, c2, w1, w2a, w2b, rest=None, wca=None, wcb=None,
               bc=None):
    cspec = pl.BlockSpec((1, M, CP), lambda b, i: (b, i, 0))
    wspec = pl.BlockSpec((LANES, LANES), lambda b, i: (0, 0))
    in_specs = [
        pl.BlockSpec((1, M, K * TW), lambda b, i: (b, i, 0)),
        cspec, cspec, cspec,
    ]
    mspec = pl.BlockSpec((1, M, CP), lambda b, i: (b, i, 0))
    mshape = jax.ShapeDtypeStruct((B, N, CP), jnp.float32)
    if rest is None:
        return pl.pallas_call(
            _edge_body,
            grid=(B, N // M),
            in_specs=in_specs + [wspec, wspec, wspec],
            out_specs=[mspec, mspec, mspec],
            out_shape=[mshape, mshape, mshape],
            compiler_params=_TC_PARAMS,
        )(ag, c0, c1, c2, w1, w2a, w2b)
    R = rest.shape[-1]
    wcaspec = pl.BlockSpec((CP, 48), lambda b, i: (0, 0))
    return pl.pallas_call(
        _edge_body_t,
        grid=(B, N // M),
        in_specs=in_specs + [
            pl.BlockSpec((1, M, R), lambda b, i: (b, i, 0)),
            wspec, wspec, wspec,
            wcaspec, wcaspec, wcaspec,
            pl.BlockSpec((R, 48), lambda b, i: (0, 0)),
            pl.BlockSpec((1, 48), lambda b, i: (0, 0)),
        ],
        out_specs=[mspec, mspec, mspec,
                   pl.BlockSpec((1, M, 48), lambda b, i: (b, i, 0))],
        out_shape=[mshape, mshape, mshape,
                   jax.ShapeDtypeStruct((B, N, 48), jnp.float32)],
        compiler_params=_TC_PARAMS,
    )(ag, c0, c1, c2, rest, w1, w2a, w2b, wca[0], wca[1], wca[2],
      wcb, bc.reshape(1, 48))


def _block(f, l_prev, W0, b0, W1, b1, W2, b2, Wc=None, bc=None):
    C = f.shape[-1]
    w0a = _pad_rc(W0[:C], C, TW)
    cw0 = _pad_rc(W0[C:] - W0[:C], C, CP)
    cw1 = _pad_rc(W1[24:], C, CP)
    cw2 = _pad_rc(W2[48:], C, CP)
    b0p = _pad_rc(b0.reshape(1, 24), 1, CP)
    b1p = _pad_rc(b1.reshape(1, 24), 1, CP)
    b2p = _pad_rc(b2.reshape(1, 24), 1, CP)

    ft = jnp.swapaxes(f, 1, 2)
    gidx, A, c0, c1, c2 = _knn_call(f, ft, w0a, cw0, cw1, cw2, b0p, b1p, b2p)

    ag = _sc_gather(A.reshape(B * N, TW), gidx.reshape(1, _NIDX))
    ag = ag.reshape(B, N, K * TW)

    eye = jnp.eye(K, dtype=jnp.float32)
    w1bd = jnp.kron(eye, _pad_rc(W1[:24], CP, CP))
    w2abd = jnp.kron(eye, _pad_rc(W2[:24], CP, CP))
    w2bbd = jnp.kron(eye, _pad_rc(W2[24:48], CP, CP))

    if Wc is None:
        m2, m1, m0 = _edge_call(ag, c0, c1, c2, w1bd, w2abd, w2bbd)
        t = None
    else:
        rest = jnp.concatenate([f, l_prev], axis=-1)
        wca = [_pad_rc(Wc[0:24], CP, 48), _pad_rc(Wc[24:48], CP, 48),
               _pad_rc(Wc[48:72], CP, 48)]
        wcb = Wc[72:]
        m2, m1, m0, t = _edge_call(ag, c0, c1, c2, w1bd, w2abd, w2bbd,
                                   rest=rest, wca=wca, wcb=wcb, bc=bc)
    maxes = jnp.concatenate([m2[..., :24], m1[..., :24], m0[..., :24]],
                            axis=-1)
    return maxes, t


def kernel(pc, W_in, b_in, Wd0_0, bd0_0, Wd0_1, bd0_1, Wd0_2, bd0_2,
           Wc1, bc1, Wd1_0, bd1_0, Wd1_1, bd1_1, Wd1_2, bd1_2,
           Wc2, bc2, Wd2_0, bd2_0, Wd2_1, bd2_1, Wd2_2, bd2_2,
           Wc3, bc3, Wd3_0, bd3_0, Wd3_1, bd3_1, Wd3_2, bd3_2):
    l0 = _l0_call(pc, W_in, b_in)

    mx1, t1 = _block(l0, l0, Wd0_0, bd0_0, Wd0_1, bd0_1, Wd0_2, bd0_2,
                     Wc=Wc1, bc=bc1)
    l1 = jnp.concatenate([mx1, l0, l0], axis=-1)

    mx2, t2 = _block(t1, l1, Wd1_0, bd1_0, Wd1_1, bd1_1, Wd1_2, bd1_2,
                     Wc=Wc2, bc=bc2)
    l2 = jnp.concatenate([mx2, t1, l1], axis=-1)

    mx3, t3 = _block(t2, l2, Wd2_0, bd2_0, Wd2_1, bd2_1, Wd2_2, bd2_2,
                     Wc=Wc3, bc=bc3)
    l3 = jnp.concatenate([mx3, t2, l2], axis=-1)

    mx4, _ = _block(t3, l3, Wd3_0, bd3_0, Wd3_1, bd3_1, Wd3_2, bd3_2)
    return jnp.concatenate([mx4, t3, l3], axis=-1)
